# baseline (speedup 1.0000x reference)
import functools

import jax
import jax.numpy as jnp
from jax import lax
from jax.experimental import pallas as pl
from jax.experimental.pallas import tpu as pltpu
from jax.experimental.pallas import tpu_sc as plsc

_INFO = plsc.get_sparse_core_info()
_NC, _NS = _INFO.num_cores, _INFO.num_subcores
_NW = _NC * _NS
_L = _INFO.num_lanes

_CHUNK = 64


def _make_sc_kernel(batch, embed, nrows, dtype):
    b_per_w = batch // _NW
    n_chunks = b_per_w // _CHUNK
    ncol = embed // _L
    mesh = plsc.VectorSubcoreMesh(core_axis_name="c", subcore_axis_name="s")

    @functools.partial(
        pl.kernel,
        out_type=jax.ShapeDtypeStruct((batch, embed), dtype),
        mesh=mesh,
        compiler_params=pltpu.CompilerParams(needs_layout_passes=False),
        scratch_types=[
            pltpu.VMEM((b_per_w,), jnp.int32),
            pltpu.VMEM((nrows, embed), dtype),
            pltpu.VMEM((_CHUNK, embed), dtype),
            pltpu.VMEM((_CHUNK, embed), dtype),
            pltpu.SemaphoreType.DMA,
            pltpu.SemaphoreType.DMA,
            pltpu.SemaphoreType.DMA,
        ],
    )
    def k(table_hbm, idx_hbm, out_hbm, idx_v, t_v, buf0, buf1, tsem, s0, s1):
        wid = lax.axis_index("s") * _NC + lax.axis_index("c")
        base = wid * b_per_w
        tfill = pltpu.async_copy(table_hbm, t_v, tsem)
        pltpu.sync_copy(idx_hbm.at[pl.ds(base, b_per_w)], idx_v)
        tfill.wait()
        bufs = (buf0, buf1)
        ssems = (s0, s1)
        lanes = lax.iota(jnp.int32, _L)
        zero16 = jnp.zeros((_L,), jnp.int32)

        def assemble(buf, c):
            def row_body(r, carry):
                sv = plsc.load_gather(idx_v, [zero16 + (c * _CHUNK + r)])
                for j in range(ncol):
                    buf[r, pl.ds(j * _L, _L)] = plsc.load_gather(
                        t_v, [sv, lanes + j * _L])
                return carry

            lax.fori_loop(0, _CHUNK, row_body, jnp.int32(0))

        for c in range(n_chunks):
            b = c & 1
            if c >= 2:
                pltpu.make_async_copy(
                    bufs[b], out_hbm.at[pl.ds(base + (c - 2) * _CHUNK, _CHUNK)],
                    ssems[b]).wait()
            assemble(bufs[b], c)
            pltpu.async_copy(
                bufs[b], out_hbm.at[pl.ds(base + c * _CHUNK, _CHUNK)],
                ssems[b])
        for c in range(max(n_chunks - 2, 0), n_chunks):
            b = c & 1
            pltpu.make_async_copy(
                bufs[b], out_hbm.at[pl.ds(base + c * _CHUNK, _CHUNK)],
                ssems[b]).wait()

    return k


def kernel(indices, table):
    batch = indices.shape[0]
    nrows, embed = table.shape
    k = _make_sc_kernel(batch, embed, nrows, table.dtype)
    return k(table, indices.astype(jnp.int32))


# speedup vs baseline: 2.5788x; 2.5788x over previous
import functools

import jax
import jax.numpy as jnp
from jax import lax
from jax.experimental import pallas as pl
from jax.experimental.pallas import tpu as pltpu
from jax.experimental.pallas import tpu_sc as plsc

_INFO = plsc.get_sparse_core_info()
_NC, _NS = _INFO.num_cores, _INFO.num_subcores
_NW = _NC * _NS
_L = _INFO.num_lanes

_CHUNK = 32
_REPL = 512


def _make_sc_kernel(batch, embed, nrows, dtype):
    b_per_w = batch // _NW
    max_chunks = b_per_w // _CHUNK
    n_groups = b_per_w // _L
    mesh = plsc.VectorSubcoreMesh(core_axis_name="c", subcore_axis_name="s")

    @functools.partial(
        pl.kernel,
        out_type=jax.ShapeDtypeStruct((batch, embed), dtype),
        mesh=mesh,
        compiler_params=pltpu.CompilerParams(needs_layout_passes=False),
        scratch_types=[
            pltpu.VMEM((b_per_w,), jnp.int32),
            pltpu.VMEM((2 * _CHUNK,), jnp.int32),
            pltpu.VMEM((b_per_w + _CHUNK,), jnp.int32),
            pltpu.VMEM((b_per_w + _CHUNK,), jnp.int32),
            pltpu.VMEM((max_chunks, _CHUNK), jnp.int32),
            pltpu.VMEM((max_chunks, _CHUNK), jnp.int32),
            pltpu.VMEM((_CHUNK, embed), dtype),
            pltpu.VMEM((_CHUNK, embed), dtype),
            pltpu.SemaphoreType.DMA,
            pltpu.SemaphoreType.DMA,
            pltpu.SemaphoreType.DMA,
        ],
    )
    def k(table_hbm, idx_hbm, out_hbm, idx_v, fill_i, p0f, p1f, p0, p1,
          src0, src1, fsem, ssem0, ssem1):
        wid = lax.axis_index("s") * _NC + lax.axis_index("c")
        base = wid * b_per_w
        lanes = lax.iota(jnp.int32, _L)

        pltpu.sync_copy(idx_hbm.at[pl.ds(base, b_per_w)], idx_v)

        for j in range(_CHUNK // _L):
            rep = (lanes + (wid * _CHUNK + j * _L)) & (_REPL - 1)
            fill_i[pl.ds(j * _L, _L)] = rep * nrows
            rep2 = (lanes + (wid * _CHUNK + j * _L + _REPL // 2)) & (_REPL - 1)
            fill_i[pl.ds(_CHUNK + j * _L, _L)] = rep2 * nrows + 1
        fill0 = pltpu.async_copy(
            table_hbm.at[fill_i.at[pl.ds(0, _CHUNK)]], src0, fsem)
        fill1 = pltpu.async_copy(
            table_hbm.at[fill_i.at[pl.ds(_CHUNK, _CHUNK)]], src1, fsem)

        n0 = jnp.int32(0)
        for j in range(n_groups):
            v = idx_v[pl.ds(j * _L, _L)]
            p = lanes + (base + j * _L)
            m0 = v == 0
            c0 = plsc.cumsum(jnp.where(m0, 1, 0))
            n1 = jnp.int32(j * _L) - n0
            plsc.store_scatter(p0f, [n0 + c0 - 1], p, mask=m0)
            plsc.store_scatter(p1f, [n1 + lanes - c0], p,
                               mask=jnp.logical_not(m0))
            n0 = n0 + jnp.max(c0)
        n1 = jnp.int32(b_per_w) - n0

        zero16 = jnp.zeros((_L,), jnp.int32)
        f0 = plsc.load_gather(p0f, [zero16])
        f1 = plsc.load_gather(p1f, [zero16])
        for t in range(_CHUNK // _L):
            p0f[pl.ds(n0 + t * _L, _L)] = f0
            p1f[pl.ds(n1 + t * _L, _L)] = f1

        for c in range(max_chunks):
            for q in range(_CHUNK // _L):
                p0[c, pl.ds(q * _L, _L)] = p0f[pl.ds(c * _CHUNK + q * _L, _L)]
                p1[c, pl.ds(q * _L, _L)] = p1f[pl.ds(c * _CHUNK + q * _L, _L)]

        k0 = lax.shift_right_logical(n0 + (_CHUNK - 1), 5)
        k1 = lax.shift_right_logical(n1 + (_CHUNK - 1), 5)

        fill0.wait()
        fill1.wait()

        for c in range(max_chunks):
            @pl.when(c < k0)
            def _s0():
                pltpu.async_copy(src0, out_hbm.at[p0.at[c]], ssem0)
            @pl.when(c < k1)
            def _s1():
                pltpu.async_copy(src1, out_hbm.at[p1.at[c]], ssem1)
        for c in range(max_chunks):
            @pl.when(c < k0)
            def _w0():
                pltpu.make_async_copy(src0, out_hbm.at[p0.at[c]], ssem0).wait()
            @pl.when(c < k1)
            def _w1():
                pltpu.make_async_copy(src1, out_hbm.at[p1.at[c]], ssem1).wait()

    return k


def kernel(indices, table):
    batch = indices.shape[0]
    nrows, embed = table.shape
    table_rep = jnp.broadcast_to(table[None, :, :], (_REPL, nrows, embed))
    table_rep = table_rep.reshape(_REPL * nrows, embed)
    k = _make_sc_kernel(batch, embed, nrows, table.dtype)
    return k(table_rep, indices.astype(jnp.int32))
